# baseline (speedup 1.0000x reference)
import functools

import jax
import jax.numpy as jnp
import numpy as np
from jax import lax
from jax.experimental import pallas as pl
from jax.experimental.pallas import tpu as pltpu
from jax.experimental.pallas import tpu_sc as plsc

_B, _N, _CIN = 2, 16384, 16
_NP = 1024
_NS0, _NS1 = 16, 32
_R0SQ = float(np.float32(0.1) * np.float32(0.1))
_R1SQ = float(np.float32(0.2) * np.float32(0.2))
_CP = 24
_L = 16
_NSUB = 16
_NLOC = _N // _NSUB
_NCHUNK = _NLOC // _L
_BIG = 0x7FFFFFFF

_mesh = plsc.VectorSubcoreMesh(core_axis_name="c", subcore_axis_name="s",
                               num_cores=2, num_subcores=16)
_sc_params = pltpu.CompilerParams(needs_layout_passes=False,
                                  use_tc_tiling_on_sc=False)


_GPTS = 128
_GCH = _GPTS // _L
_NGRP = _N // _GPTS
_CENT_PER_SUB = _NP // _NSUB


def _fused_body(xs, ys, zs, table2d, cpad, g0, g1, xv, yv, zv, dv, rowv, bufv,
                accv, centv, acc0, acc1, rows, shared, shcent, gsem, wsem):
    c = lax.axis_index("c")
    s = lax.axis_index("s")
    base = s * _NLOC
    pltpu.sync_copy(xs.at[pl.ds(c * _N, _N)], xv)
    pltpu.sync_copy(ys.at[pl.ds(c * _N, _N)], yv)
    pltpu.sync_copy(zs.at[pl.ds(c * _N, _N)], zv)
    lanes = lax.iota(jnp.int32, _L)
    z16 = jnp.zeros((_L,), jnp.int32)
    ninf = jnp.float32(-1e30)

    def init_d(j, _):
        dv[pl.ds(j * _L, _L)] = jnp.full((_L,), 1e10, jnp.float32)
        return 0
    lax.fori_loop(0, _NCHUNK, init_d, 0)

    def publish_reduce(d_s, gi_s, par):
        row = jnp.where(lanes == 0, d_s,
              jnp.where(lanes == 1, gi_s.astype(jnp.float32), 0.0))
        rowv[...] = row
        pb = par * (_L * _L)
        pltpu.sync_copy(rowv, shared.at[pl.ds(pb + s * _L, _L)])
        plsc.subcore_barrier()

        @pl.when(s == 0)
        def _():
            pltpu.sync_copy(shared.at[pl.ds(pb, _L * _L)], bufv)
            tcol = lanes * _L
            dcol = plsc.load_gather(bufv, [tcol])
            gmax = jnp.max(dcol)
            wl = plsc.all_reduce_ffs(dcol == gmax)
            giv = plsc.load_gather(bufv, [wl * _L + 1])
            gii = giv.astype(jnp.int32)
            res = jnp.where(lanes == 0, giv,
                  jnp.where(lanes == 1, plsc.load_gather(xv, [gii]),
                  jnp.where(lanes == 2, plsc.load_gather(yv, [gii]),
                  jnp.where(lanes == 3, plsc.load_gather(zv, [gii]), 0.0))))
            rowv[...] = res
            pltpu.sync_copy(rowv, shared.at[pl.ds(2 * _L * _L + par * _L, _L)])
        plsc.subcore_barrier()
        pltpu.sync_copy(shared.at[pl.ds(2 * _L * _L + par * _L, _L)],
                        bufv.at[pl.ds(0, _L)])
        cxw = plsc.load_gather(bufv, [z16 + 1])
        cyw = plsc.load_gather(bufv, [z16 + 2])
        czw = plsc.load_gather(bufv, [z16 + 3])
        return cxw, cyw, czw

    d0 = jnp.where(s == 0, jnp.float32(1e30), ninf)
    carry0 = publish_reduce(d0, jnp.int32(0), jnp.int32(0))

    def body(i, carry):
        cx, cy, cz = carry

        @pl.when(s == 0)
        def _():
            rowc = jnp.where(lanes == 0, cx,
                   jnp.where(lanes == 1, cy,
                   jnp.where(lanes == 2, cz, 0.0)))
            plsc.store_scatter(accv, [i * _L + lanes], rowc)

        def chunk(j, ch):
            bd, bi = ch
            off = base + j * _L
            dx = xv[pl.ds(off, _L)] - cx
            dy = yv[pl.ds(off, _L)] - cy
            dz = zv[pl.ds(off, _L)] - cz
            d = (dx * dx + dy * dy) + dz * dz
            dmin = jnp.minimum(dv[pl.ds(j * _L, _L)], d)
            dv[pl.ds(j * _L, _L)] = dmin
            upd = dmin > bd
            bd = jnp.where(upd, dmin, bd)
            bi = jnp.where(upd, off + lanes, bi)
            return bd, bi

        bd, bi = plsc.parallel_loop(
            0, _NCHUNK, unroll=4,
            carry=(jnp.full((_L,), -1.0, jnp.float32),
                   jnp.zeros((_L,), jnp.int32)))(chunk)
        md = jnp.max(bd)
        gidx = jnp.min(jnp.where(bd == md, bi, _BIG))
        return publish_reduce(md, gidx, (i + 1) & 1)

    lax.fori_loop(0, _NP, body, carry0)

    @pl.when(s == 0)
    def _():
        pltpu.sync_copy(accv, cpad.at[pl.ds(c * _NP * _L, _NP * _L)])
        pltpu.sync_copy(accv, shcent)
    plsc.subcore_barrier()
    pltpu.sync_copy(
        shcent.at[pl.ds(s * _CENT_PER_SUB * _L, _CENT_PER_SUB * _L)], centv)

    b = c
    roff = b * _N

    def per_cent(ci, _):
        crow = centv[pl.ds(ci * _L, _L)]
        cx = jnp.max(jnp.where(lanes == 0, crow, ninf))
        cy = jnp.max(jnp.where(lanes == 1, crow, ninf))
        cz = jnp.max(jnp.where(lanes == 2, crow, ninf))

        def cond(st):
            g, c0, c1, notdone = st
            return (g < _NGRP) & notdone

        def wbody(st):
            g, c0, c1, notdone = st
            m0s, m1s = [], []
            for j in range(_GCH):
                off = g * _GPTS + j * _L
                dx = xv[pl.ds(off, _L)] - cx
                dy = yv[pl.ds(off, _L)] - cy
                dz = zv[pl.ds(off, _L)] - cz
                sq = (dx * dx + dy * dy) + dz * dz
                m0s.append(sq < _R0SQ)
                m1s.append(sq < _R1SQ)
            m1or = functools.reduce(jnp.bitwise_or, m1s)
            m0or = functools.reduce(jnp.bitwise_or, m0s)
            mm = (m1or & (c1 < _NS1)) | (m0or & (c0 < _NS0))

            def slow(_):
                cc0, cc1 = c0, c1
                for j in range(_GCH):
                    gv = (roff + g * _GPTS + j * _L) + lanes
                    p0 = cc0 + plsc.cumsum(m0s[j].astype(jnp.int32)) - 1
                    plsc.store_scatter(acc0, [p0, z16 + ci], gv,
                                       mask=m0s[j] & (p0 < _NS0))
                    cc0 = cc0 + plsc.all_reduce_population_count(m0s[j])
                    p1 = cc1 + plsc.cumsum(m1s[j].astype(jnp.int32)) - 1
                    plsc.store_scatter(acc1, [p1, z16 + ci], gv,
                                       mask=m1s[j] & (p1 < _NS1))
                    cc1 = cc1 + plsc.all_reduce_population_count(m1s[j])
                nd = jnp.any((cc0 < _NS0) | (cc1 < _NS1))
                return cc0, cc1, nd

            c0n, c1n, ndn = lax.cond(jnp.any(mm), slow,
                                     lambda _: (c0, c1, notdone), 0)
            return g + jnp.int32(1), c0n, c1n, ndn

        zc = jnp.zeros((_L,), jnp.int32)
        g, c0, c1, _nd = lax.while_loop(
            cond, wbody, (jnp.int32(0), zc, zc, jnp.bool_(True)))
        civ = z16 + ci
        f0 = plsc.load_gather(acc0, [z16, civ])
        f0 = jnp.where(c0 > 0, f0, roff)
        plsc.store_scatter(acc0, [lanes, civ], f0, mask=lanes >= c0)
        f1 = plsc.load_gather(acc1, [z16, civ])
        f1 = jnp.where(c1 > 0, f1, roff)
        plsc.store_scatter(acc1, [lanes, civ], f1, mask=lanes >= c1)
        plsc.store_scatter(acc1, [lanes + _L, civ], f1, mask=(lanes + _L) >= c1)
        return 0

    lax.fori_loop(0, _CENT_PER_SUB, per_cent, 0)

    pbase = b * _NP + s * _CENT_PER_SUB

    items = [(acc0, g0, sl) for sl in range(_NS0)] \
          + [(acc1, g1, sl) for sl in range(_NS1)]

    def fire(i):
        accr, gh, sl = items[i]
        return pltpu.async_copy(table2d.at[accr.at[sl]], rows.at[i & 1], gsem)

    gd = [fire(0), None]
    wd = [None, None]
    for i in range(len(items)):
        p = i & 1
        _, gh, sl = items[i]
        gd[p].wait()
        if i + 1 < len(items):
            q = (i + 1) & 1
            if wd[q] is not None:
                wd[q].wait()
                wd[q] = None
            gd[q] = fire(i + 1)
        wd[p] = pltpu.async_copy(
            rows.at[p], gh.at[sl, pl.ds(pbase, _CENT_PER_SUB)], wsem)
    for d in wd:
        if d is not None:
            d.wait()


_fused_call = functools.partial(
    pl.kernel, _fused_body,
    out_type=(jax.ShapeDtypeStruct((_B * _NP * _L,), jnp.float32),
              jax.ShapeDtypeStruct((_NS0, _B * _NP, _CP), jnp.float32),
              jax.ShapeDtypeStruct((_NS1, _B * _NP, _CP), jnp.float32)),
    mesh=_mesh,
    scratch_types=[
        pltpu.VMEM((_N,), jnp.float32),
        pltpu.VMEM((_N,), jnp.float32),
        pltpu.VMEM((_N,), jnp.float32),
        pltpu.VMEM((_NLOC,), jnp.float32),
        pltpu.VMEM((_L,), jnp.float32),
        pltpu.VMEM((_L * _L,), jnp.float32),
        pltpu.VMEM((_NP * _L,), jnp.float32),
        pltpu.VMEM((_CENT_PER_SUB * _L,), jnp.float32),
        pltpu.VMEM((_NS0, _CENT_PER_SUB), jnp.int32),
        pltpu.VMEM((_NS1, _CENT_PER_SUB), jnp.int32),
        pltpu.VMEM((2, _CENT_PER_SUB, _CP), jnp.float32),
        pltpu.VMEM_SHARED((2 * _L * _L + 2 * _L,), jnp.float32),
        pltpu.VMEM_SHARED((_NP * _L,), jnp.float32),
        pltpu.SemaphoreType.DMA,
        pltpu.SemaphoreType.DMA,
    ],
    compiler_params=_sc_params)()


def _mlp_body(g0, g1, nx, w00, b00, w01, b01, w10, b10, w11, b11, o0, o1):
    s = pl.program_id(0)
    nxv = nx[...]

    def two_layer(g_ref, w1, bb1, w2, bb2):
        h = g_ref[0] - nxv
        a = jnp.dot(h, w1[...], preferred_element_type=jnp.float32) + bb1[...]
        a = jnp.maximum(a, 0.0)
        a = jnp.dot(a, w2[...], preferred_element_type=jnp.float32) + bb2[...]
        return jnp.maximum(a, 0.0)

    a1 = two_layer(g1, w10, b10, w11, b11)

    @pl.when(s == 0)
    def _():
        o1[...] = a1

    @pl.when(s > 0)
    def _():
        o1[...] = jnp.maximum(o1[...], a1)

    @pl.when(s < _NS0)
    def _():
        a0 = two_layer(g0, w00, b00, w01, b01)

        @pl.when(s == 0)
        def _():
            o0[...] = a0

        @pl.when(s > 0)
        def _():
            o0[...] = jnp.maximum(o0[...], a0)


def _mlp_call(g0, g1, nxyz24, w00, b00, w01, b01, w10, b10, w11, b11):
    r = _B * _NP
    full = lambda shape: pl.BlockSpec(shape, lambda s: (0,) * len(shape))
    return pl.pallas_call(
        _mlp_body,
        grid=(_NS1,),
        in_specs=[
            pl.BlockSpec((1, r, _CP), lambda s: (jnp.minimum(s, _NS0 - 1), 0, 0)),
            pl.BlockSpec((1, r, _CP), lambda s: (s, 0, 0)),
            full((r, _CP)),
            full((_CP, 32)), full((1, 32)),
            full((32, 32)), full((1, 32)),
            full((_CP, 32)), full((1, 32)),
            full((32, 64)), full((1, 64)),
        ],
        out_specs=[full((r, 32)), full((r, 64))],
        out_shape=[jax.ShapeDtypeStruct((r, 32), jnp.float32),
                   jax.ShapeDtypeStruct((r, 64), jnp.float32)],
    )(g0, g1, nxyz24, w00, b00, w01, b01, w10, b10, w11, b11)


def kernel(xyz, features, W0_0, b0_0, W0_1, b0_1, W1_0, b1_0, W1_1, b1_1):
    xs = xyz[:, :, 0].reshape(-1)
    ys = xyz[:, :, 1].reshape(-1)
    zs = xyz[:, :, 2].reshape(-1)
    table = jnp.concatenate(
        [xyz, jnp.transpose(features, (0, 2, 1)),
         jnp.zeros((_B, _N, _CP - 3 - _CIN), jnp.float32)], axis=-1)
    table2d = table.reshape(_B * _N, _CP)
    cpad_flat, g0, g1 = _fused_call(xs, ys, zs, table2d)
    cpad = cpad_flat.reshape(_B, _NP, _L)
    nxyz24 = jnp.pad(cpad.reshape(_B * _NP, _L), ((0, 0), (0, _CP - _L)))
    pad_w = lambda w: jnp.pad(w.T, ((0, _CP - w.shape[1]), (0, 0)))
    out0, out1 = _mlp_call(
        g0, g1, nxyz24,
        pad_w(W0_0), b0_0.reshape(1, -1), W0_1.T, b0_1.reshape(1, -1),
        pad_w(W1_0), b1_0.reshape(1, -1), W1_1.T, b1_1.reshape(1, -1))
    new_xyz = cpad[:, :, :3]
    nf0 = jnp.transpose(out0.reshape(_B, _NP, 32), (0, 2, 1))
    nf1 = jnp.transpose(out1.reshape(_B, _NP, 64), (0, 2, 1))
    new_features = jnp.concatenate([nf0, nf1], axis=1)
    return new_xyz, new_features


# speedup vs baseline: 1.2556x; 1.2556x over previous
import functools

import jax
import jax.numpy as jnp
import numpy as np
from jax import lax
from jax.experimental import pallas as pl
from jax.experimental.pallas import tpu as pltpu
from jax.experimental.pallas import tpu_sc as plsc

_B, _N, _CIN = 2, 16384, 16
_NP = 1024
_NS0, _NS1 = 16, 32
_R0SQ = float(np.float32(0.1) * np.float32(0.1))
_R1SQ = float(np.float32(0.2) * np.float32(0.2))
_CP = 24
_L = 16
_NSUB = 16
_NLOC = _N // _NSUB
_NCHUNK = _NLOC // _L
_BIG = 0x7FFFFFFF

_mesh = plsc.VectorSubcoreMesh(core_axis_name="c", subcore_axis_name="s",
                               num_cores=2, num_subcores=16)
_sc_params = pltpu.CompilerParams(needs_layout_passes=False,
                                  use_tc_tiling_on_sc=False)


_GPTS = 128
_GCH = _GPTS // _L
_NGRP = _N // _GPTS
_CENT_PER_SUB = _NP // _NSUB


def _fused_body(xs, ys, zs, table2d, cpad, g0, g1, xv, yv, zv, dv, rowv, bufv,
                accv, centv, acc0, acc1, rows, shared, shcent, gsem, wsem):
    c = lax.axis_index("c")
    s = lax.axis_index("s")
    base = s * _NLOC
    pltpu.sync_copy(xs.at[pl.ds(c * _N, _N)], xv)
    pltpu.sync_copy(ys.at[pl.ds(c * _N, _N)], yv)
    pltpu.sync_copy(zs.at[pl.ds(c * _N, _N)], zv)
    lanes = lax.iota(jnp.int32, _L)
    z16 = jnp.zeros((_L,), jnp.int32)
    ninf = jnp.float32(-1e30)

    def init_d(j, _):
        dv[pl.ds(j * _L, _L)] = jnp.full((_L,), 1e10, jnp.float32)
        return 0
    lax.fori_loop(0, _NCHUNK, init_d, 0)

    def publish_reduce(d_s, gi_s, par):
        row = jnp.where(lanes == 0, d_s,
              jnp.where(lanes == 1, gi_s.astype(jnp.float32), 0.0))
        rowv[...] = row
        pb = par * (_L * _L)
        pltpu.sync_copy(rowv, shared.at[pl.ds(pb + s * _L, _L)])
        plsc.subcore_barrier()
        pltpu.sync_copy(shared.at[pl.ds(pb, _L * _L)], bufv)
        tcol = lanes * _L
        dcol = plsc.load_gather(bufv, [tcol])
        gmax = jnp.max(dcol)
        wl = plsc.all_reduce_ffs(dcol == gmax)
        giv = plsc.load_gather(bufv, [wl * _L + 1]).astype(jnp.int32)
        cxw = plsc.load_gather(xv, [giv])
        cyw = plsc.load_gather(yv, [giv])
        czw = plsc.load_gather(zv, [giv])
        return cxw, cyw, czw

    d0 = jnp.where(s == 0, jnp.float32(1e30), ninf)
    carry0 = publish_reduce(d0, jnp.int32(0), jnp.int32(0))

    def body(i, carry):
        cx, cy, cz = carry

        @pl.when(s == 0)
        def _():
            rowc = jnp.where(lanes == 0, cx,
                   jnp.where(lanes == 1, cy,
                   jnp.where(lanes == 2, cz, 0.0)))
            plsc.store_scatter(accv, [i * _L + lanes], rowc)

        def chunk(j, ch):
            bd, bi = ch
            off = base + j * _L
            dx = xv[pl.ds(off, _L)] - cx
            dy = yv[pl.ds(off, _L)] - cy
            dz = zv[pl.ds(off, _L)] - cz
            d = (dx * dx + dy * dy) + dz * dz
            dmin = jnp.minimum(dv[pl.ds(j * _L, _L)], d)
            dv[pl.ds(j * _L, _L)] = dmin
            upd = dmin > bd
            bd = jnp.where(upd, dmin, bd)
            bi = jnp.where(upd, off + lanes, bi)
            return bd, bi

        bd, bi = plsc.parallel_loop(
            0, _NCHUNK, unroll=4,
            carry=(jnp.full((_L,), -1.0, jnp.float32),
                   jnp.zeros((_L,), jnp.int32)))(chunk)
        md = jnp.max(bd)
        gidx = jnp.min(jnp.where(bd == md, bi, _BIG))
        return publish_reduce(md, gidx, (i + 1) & 1)

    lax.fori_loop(0, _NP, body, carry0)

    @pl.when(s == 0)
    def _():
        pltpu.sync_copy(accv, cpad.at[pl.ds(c * _NP * _L, _NP * _L)])
        pltpu.sync_copy(accv, shcent)
    plsc.subcore_barrier()
    pltpu.sync_copy(
        shcent.at[pl.ds(s * _CENT_PER_SUB * _L, _CENT_PER_SUB * _L)], centv)

    b = c
    roff = b * _N

    def per_cent(ci, _):
        crow = centv[pl.ds(ci * _L, _L)]
        cx = jnp.max(jnp.where(lanes == 0, crow, ninf))
        cy = jnp.max(jnp.where(lanes == 1, crow, ninf))
        cz = jnp.max(jnp.where(lanes == 2, crow, ninf))

        def cond(st):
            g, c0, c1, notdone = st
            return (g < _NGRP) & notdone

        def wbody(st):
            g, c0, c1, notdone = st
            m0s, m1s = [], []
            for j in range(_GCH):
                off = g * _GPTS + j * _L
                dx = xv[pl.ds(off, _L)] - cx
                dy = yv[pl.ds(off, _L)] - cy
                dz = zv[pl.ds(off, _L)] - cz
                sq = (dx * dx + dy * dy) + dz * dz
                m0s.append(sq < _R0SQ)
                m1s.append(sq < _R1SQ)
            m1or = functools.reduce(jnp.bitwise_or, m1s)
            m0or = functools.reduce(jnp.bitwise_or, m0s)
            mm = (m1or & (c1 < _NS1)) | (m0or & (c0 < _NS0))

            def slow(_):
                cc0, cc1 = c0, c1
                for j in range(_GCH):
                    gv = (roff + g * _GPTS + j * _L) + lanes
                    p0 = cc0 + plsc.cumsum(m0s[j].astype(jnp.int32)) - 1
                    plsc.store_scatter(acc0, [p0, z16 + ci], gv,
                                       mask=m0s[j] & (p0 < _NS0))
                    cc0 = cc0 + plsc.all_reduce_population_count(m0s[j])
                    p1 = cc1 + plsc.cumsum(m1s[j].astype(jnp.int32)) - 1
                    plsc.store_scatter(acc1, [p1, z16 + ci], gv,
                                       mask=m1s[j] & (p1 < _NS1))
                    cc1 = cc1 + plsc.all_reduce_population_count(m1s[j])
                nd = jnp.any((cc0 < _NS0) | (cc1 < _NS1))
                return cc0, cc1, nd

            c0n, c1n, ndn = lax.cond(jnp.any(mm), slow,
                                     lambda _: (c0, c1, notdone), 0)
            return g + jnp.int32(1), c0n, c1n, ndn

        zc = jnp.zeros((_L,), jnp.int32)
        g, c0, c1, _nd = lax.while_loop(
            cond, wbody, (jnp.int32(0), zc, zc, jnp.bool_(True)))
        civ = z16 + ci
        f0 = plsc.load_gather(acc0, [z16, civ])
        f0 = jnp.where(c0 > 0, f0, roff)
        plsc.store_scatter(acc0, [lanes, civ], f0, mask=lanes >= c0)
        f1 = plsc.load_gather(acc1, [z16, civ])
        f1 = jnp.where(c1 > 0, f1, roff)
        plsc.store_scatter(acc1, [lanes, civ], f1, mask=lanes >= c1)
        plsc.store_scatter(acc1, [lanes + _L, civ], f1, mask=(lanes + _L) >= c1)
        return 0

    lax.fori_loop(0, _CENT_PER_SUB, per_cent, 0)

    pbase = b * _NP + s * _CENT_PER_SUB

    items = [(acc0, g0, sl) for sl in range(_NS0)] \
          + [(acc1, g1, sl) for sl in range(_NS1)]

    def fire(i):
        accr, gh, sl = items[i]
        return pltpu.async_copy(table2d.at[accr.at[sl]], rows.at[i & 1], gsem)

    gd = [fire(0), None]
    wd = [None, None]
    for i in range(len(items)):
        p = i & 1
        _, gh, sl = items[i]
        gd[p].wait()
        if i + 1 < len(items):
            q = (i + 1) & 1
            if wd[q] is not None:
                wd[q].wait()
                wd[q] = None
            gd[q] = fire(i + 1)
        wd[p] = pltpu.async_copy(
            rows.at[p], gh.at[sl, pl.ds(pbase, _CENT_PER_SUB)], wsem)
    for d in wd:
        if d is not None:
            d.wait()


_fused_call = functools.partial(
    pl.kernel, _fused_body,
    out_type=(jax.ShapeDtypeStruct((_B * _NP * _L,), jnp.float32),
              jax.ShapeDtypeStruct((_NS0, _B * _NP, _CP), jnp.float32),
              jax.ShapeDtypeStruct((_NS1, _B * _NP, _CP), jnp.float32)),
    mesh=_mesh,
    scratch_types=[
        pltpu.VMEM((_N,), jnp.float32),
        pltpu.VMEM((_N,), jnp.float32),
        pltpu.VMEM((_N,), jnp.float32),
        pltpu.VMEM((_NLOC,), jnp.float32),
        pltpu.VMEM((_L,), jnp.float32),
        pltpu.VMEM((_L * _L,), jnp.float32),
        pltpu.VMEM((_NP * _L,), jnp.float32),
        pltpu.VMEM((_CENT_PER_SUB * _L,), jnp.float32),
        pltpu.VMEM((_NS0, _CENT_PER_SUB), jnp.int32),
        pltpu.VMEM((_NS1, _CENT_PER_SUB), jnp.int32),
        pltpu.VMEM((2, _CENT_PER_SUB, _CP), jnp.float32),
        pltpu.VMEM_SHARED((2 * _L * _L + 2 * _L,), jnp.float32),
        pltpu.VMEM_SHARED((_NP * _L,), jnp.float32),
        pltpu.SemaphoreType.DMA,
        pltpu.SemaphoreType.DMA,
    ],
    compiler_params=_sc_params)()


_MGRID = 4
_S1B = _NS1 // _MGRID
_S0B = _NS0 // _MGRID


def _mlp_body(g0, g1, nx, w00, b00, w01, b01, w10, b10, w11, b11, o0, o1):
    s = pl.program_id(0)
    nxv = nx[...]

    def two_layer(g_ref, sl, w1, bb1, w2, bb2):
        h = g_ref[sl] - nxv
        a = jnp.dot(h, w1[...], preferred_element_type=jnp.float32) + bb1[...]
        a = jnp.maximum(a, 0.0)
        a = jnp.dot(a, w2[...], preferred_element_type=jnp.float32) + bb2[...]
        return jnp.maximum(a, 0.0)

    a1 = two_layer(g1, 0, w10, b10, w11, b11)
    for sl in range(1, _S1B):
        a1 = jnp.maximum(a1, two_layer(g1, sl, w10, b10, w11, b11))

    @pl.when(s == 0)
    def _():
        o1[...] = a1

    @pl.when(s > 0)
    def _():
        o1[...] = jnp.maximum(o1[...], a1)

    a0 = two_layer(g0, 0, w00, b00, w01, b01)
    for sl in range(1, _S0B):
        a0 = jnp.maximum(a0, two_layer(g0, sl, w00, b00, w01, b01))

    @pl.when(s == 0)
    def _():
        o0[...] = a0

    @pl.when(s > 0)
    def _():
        o0[...] = jnp.maximum(o0[...], a0)


def _mlp_call(g0, g1, nxyz24, w00, b00, w01, b01, w10, b10, w11, b11):
    r = _B * _NP
    full = lambda shape: pl.BlockSpec(shape, lambda s: (0,) * len(shape))
    return pl.pallas_call(
        _mlp_body,
        grid=(_MGRID,),
        in_specs=[
            pl.BlockSpec((_S0B, r, _CP), lambda s: (s, 0, 0)),
            pl.BlockSpec((_S1B, r, _CP), lambda s: (s, 0, 0)),
            full((r, _CP)),
            full((_CP, 32)), full((1, 32)),
            full((32, 32)), full((1, 32)),
            full((_CP, 32)), full((1, 32)),
            full((32, 64)), full((1, 64)),
        ],
        out_specs=[full((r, 32)), full((r, 64))],
        out_shape=[jax.ShapeDtypeStruct((r, 32), jnp.float32),
                   jax.ShapeDtypeStruct((r, 64), jnp.float32)],
    )(g0, g1, nxyz24, w00, b00, w01, b01, w10, b10, w11, b11)


def kernel(xyz, features, W0_0, b0_0, W0_1, b0_1, W1_0, b1_0, W1_1, b1_1):
    xs = xyz[:, :, 0].reshape(-1)
    ys = xyz[:, :, 1].reshape(-1)
    zs = xyz[:, :, 2].reshape(-1)
    table = jnp.concatenate(
        [xyz, jnp.transpose(features, (0, 2, 1)),
         jnp.zeros((_B, _N, _CP - 3 - _CIN), jnp.float32)], axis=-1)
    table2d = table.reshape(_B * _N, _CP)
    cpad_flat, g0, g1 = _fused_call(xs, ys, zs, table2d)
    cpad = cpad_flat.reshape(_B, _NP, _L)
    nxyz24 = jnp.pad(cpad.reshape(_B * _NP, _L), ((0, 0), (0, _CP - _L)))
    pad_w = lambda w: jnp.pad(w.T, ((0, _CP - w.shape[1]), (0, 0)))
    out0, out1 = _mlp_call(
        g0, g1, nxyz24,
        pad_w(W0_0), b0_0.reshape(1, -1), W0_1.T, b0_1.reshape(1, -1),
        pad_w(W1_0), b1_0.reshape(1, -1), W1_1.T, b1_1.reshape(1, -1))
    new_xyz = cpad[:, :, :3]
    nf0 = jnp.transpose(out0.reshape(_B, _NP, 32), (0, 2, 1))
    nf1 = jnp.transpose(out1.reshape(_B, _NP, 64), (0, 2, 1))
    new_features = jnp.concatenate([nf0, nf1], axis=1)
    return new_xyz, new_features


# speedup vs baseline: 1.2773x; 1.0173x over previous
import functools

import jax
import jax.numpy as jnp
import numpy as np
from jax import lax
from jax.experimental import pallas as pl
from jax.experimental.pallas import tpu as pltpu
from jax.experimental.pallas import tpu_sc as plsc

_B, _N, _CIN = 2, 16384, 16
_NP = 1024
_NS0, _NS1 = 16, 32
_R0SQ = float(np.float32(0.1) * np.float32(0.1))
_R1SQ = float(np.float32(0.2) * np.float32(0.2))
_CP = 24
_L = 16
_NSUB = 16
_NLOC = _N // _NSUB
_NCHUNK = _NLOC // _L
_BIG = 0x7FFFFFFF

_mesh = plsc.VectorSubcoreMesh(core_axis_name="c", subcore_axis_name="s",
                               num_cores=2, num_subcores=16)
_sc_params = pltpu.CompilerParams(needs_layout_passes=False,
                                  use_tc_tiling_on_sc=False)


def _fps_body(xs, ys, zs, cpad, xv, yv, zv, dv, rowv, bufv, accv, shared):
    c = lax.axis_index("c")
    s = lax.axis_index("s")
    base = s * _NLOC
    pltpu.sync_copy(xs.at[pl.ds(c * _N, _N)], xv)
    pltpu.sync_copy(ys.at[pl.ds(c * _N, _N)], yv)
    pltpu.sync_copy(zs.at[pl.ds(c * _N, _N)], zv)
    lanes = lax.iota(jnp.int32, _L)
    z16 = jnp.zeros((_L,), jnp.int32)
    ninf = jnp.float32(-1e30)

    def init_d(j, _):
        dv[pl.ds(j * _L, _L)] = jnp.full((_L,), 1e10, jnp.float32)
        return 0
    lax.fori_loop(0, _NCHUNK, init_d, 0)

    def publish_reduce(d_s, gi_s, par):
        row = jnp.where(lanes == 0, d_s,
              jnp.where(lanes == 1, gi_s.astype(jnp.float32), 0.0))
        rowv[...] = row
        pb = par * (_L * _L)
        pltpu.sync_copy(rowv, shared.at[pl.ds(pb + s * _L, _L)])
        plsc.subcore_barrier()
        pltpu.sync_copy(shared.at[pl.ds(pb, _L * _L)], bufv)
        tcol = lanes * _L
        dcol = plsc.load_gather(bufv, [tcol])
        gmax = jnp.max(dcol)
        wl = plsc.all_reduce_ffs(dcol == gmax)
        giv = plsc.load_gather(bufv, [wl * _L + 1]).astype(jnp.int32)
        cxw = plsc.load_gather(xv, [giv])
        cyw = plsc.load_gather(yv, [giv])
        czw = plsc.load_gather(zv, [giv])
        return cxw, cyw, czw

    d0 = jnp.where(s == 0, jnp.float32(1e30), ninf)
    carry0 = publish_reduce(d0, jnp.int32(0), jnp.int32(0))

    def body(i, carry):
        cx, cy, cz = carry

        @pl.when(s == 0)
        def _():
            rowc = jnp.where(lanes == 0, cx,
                   jnp.where(lanes == 1, cy,
                   jnp.where(lanes == 2, cz, 0.0)))
            plsc.store_scatter(accv, [i * _L + lanes], rowc)

        def chunk(j, ch):
            bd, bi = ch
            off = base + j * _L
            dx = xv[pl.ds(off, _L)] - cx
            dy = yv[pl.ds(off, _L)] - cy
            dz = zv[pl.ds(off, _L)] - cz
            d = (dx * dx + dy * dy) + dz * dz
            dmin = jnp.minimum(dv[pl.ds(j * _L, _L)], d)
            dv[pl.ds(j * _L, _L)] = dmin
            upd = dmin > bd
            bd = jnp.where(upd, dmin, bd)
            bi = jnp.where(upd, off + lanes, bi)
            return bd, bi

        bd, bi = plsc.parallel_loop(
            0, _NCHUNK, unroll=4,
            carry=(jnp.full((_L,), -1.0, jnp.float32),
                   jnp.zeros((_L,), jnp.int32)))(chunk)
        md = jnp.max(bd)
        gidx = jnp.min(jnp.where(bd == md, bi, _BIG))
        return publish_reduce(md, gidx, (i + 1) & 1)

    lax.fori_loop(0, _NP, body, carry0)

    @pl.when(s == 0)
    def _():
        pltpu.sync_copy(accv, cpad.at[pl.ds(c * _NP * _L, _NP * _L)])


_fps_call = functools.partial(
    pl.kernel, _fps_body,
    out_type=jax.ShapeDtypeStruct((_B * _NP * _L,), jnp.float32),
    mesh=_mesh,
    scratch_types=[
        pltpu.VMEM((_N,), jnp.float32),
        pltpu.VMEM((_N,), jnp.float32),
        pltpu.VMEM((_N,), jnp.float32),
        pltpu.VMEM((_NLOC,), jnp.float32),
        pltpu.VMEM((_L,), jnp.float32),
        pltpu.VMEM((_L * _L,), jnp.float32),
        pltpu.VMEM((_NP * _L,), jnp.float32),
        pltpu.VMEM_SHARED((2 * _L * _L,), jnp.float32),
    ],
    compiler_params=_sc_params)()


_GPTS = 128
_GCH = _GPTS // _L
_NGRP = _N // _GPTS
_CENT_PER_SUB = _NP // _NSUB


def _bq_body(xs, ys, zs, cpadf, table2d, g0, g1, xv, yv, zv, centv, acc0, acc1,
             rows, gsem, wsem):
    c = lax.axis_index("c")
    s = lax.axis_index("s")
    b = c
    pltpu.sync_copy(xs.at[pl.ds(b * _N, _N)], xv)
    pltpu.sync_copy(ys.at[pl.ds(b * _N, _N)], yv)
    pltpu.sync_copy(zs.at[pl.ds(b * _N, _N)], zv)
    cb = b * _NP * _L + s * _CENT_PER_SUB * _L
    pltpu.sync_copy(cpadf.at[pl.ds(cb, _CENT_PER_SUB * _L)], centv)
    lanes = lax.iota(jnp.int32, _L)
    z16 = jnp.zeros((_L,), jnp.int32)
    ninf = jnp.float32(-1e30)
    roff = b * _N

    def per_cent(ci, _):
        crow = centv[pl.ds(ci * _L, _L)]
        cx = jnp.max(jnp.where(lanes == 0, crow, ninf))
        cy = jnp.max(jnp.where(lanes == 1, crow, ninf))
        cz = jnp.max(jnp.where(lanes == 2, crow, ninf))

        def cond(st):
            g, c0, c1, notdone = st
            return (g < _NGRP) & notdone

        def wbody(st):
            g, c0, c1, notdone = st
            m0s, m1s = [], []
            for j in range(_GCH):
                off = g * _GPTS + j * _L
                dx = xv[pl.ds(off, _L)] - cx
                dy = yv[pl.ds(off, _L)] - cy
                dz = zv[pl.ds(off, _L)] - cz
                sq = (dx * dx + dy * dy) + dz * dz
                m0s.append(sq < _R0SQ)
                m1s.append(sq < _R1SQ)
            m1or = functools.reduce(jnp.bitwise_or, m1s)
            m0or = functools.reduce(jnp.bitwise_or, m0s)
            mm = (m1or & (c1 < _NS1)) | (m0or & (c0 < _NS0))

            def slow(_):
                cc0, cc1 = c0, c1
                for j in range(_GCH):
                    gv = (roff + g * _GPTS + j * _L) + lanes
                    p0 = cc0 + plsc.cumsum(m0s[j].astype(jnp.int32)) - 1
                    plsc.store_scatter(acc0, [p0, z16 + ci], gv,
                                       mask=m0s[j] & (p0 < _NS0))
                    cc0 = cc0 + plsc.all_reduce_population_count(m0s[j])
                    p1 = cc1 + plsc.cumsum(m1s[j].astype(jnp.int32)) - 1
                    plsc.store_scatter(acc1, [p1, z16 + ci], gv,
                                       mask=m1s[j] & (p1 < _NS1))
                    cc1 = cc1 + plsc.all_reduce_population_count(m1s[j])
                nd = jnp.any((cc0 < _NS0) | (cc1 < _NS1))
                return cc0, cc1, nd

            c0n, c1n, ndn = lax.cond(jnp.any(mm), slow,
                                     lambda _: (c0, c1, notdone), 0)
            return g + jnp.int32(1), c0n, c1n, ndn

        zc = jnp.zeros((_L,), jnp.int32)
        g, c0, c1, _nd = lax.while_loop(
            cond, wbody, (jnp.int32(0), zc, zc, jnp.bool_(True)))
        civ = z16 + ci
        f0 = plsc.load_gather(acc0, [z16, civ])
        f0 = jnp.where(c0 > 0, f0, roff)
        plsc.store_scatter(acc0, [lanes, civ], f0, mask=lanes >= c0)
        f1 = plsc.load_gather(acc1, [z16, civ])
        f1 = jnp.where(c1 > 0, f1, roff)
        plsc.store_scatter(acc1, [lanes, civ], f1, mask=lanes >= c1)
        plsc.store_scatter(acc1, [lanes + _L, civ], f1, mask=(lanes + _L) >= c1)
        return 0

    lax.fori_loop(0, _CENT_PER_SUB, per_cent, 0)

    pbase = b * _NP + s * _CENT_PER_SUB

    items = [(acc0, g0, sl) for sl in range(_NS0)] \
          + [(acc1, g1, sl) for sl in range(_NS1)]

    def fire(i):
        accr, gh, sl = items[i]
        return pltpu.async_copy(table2d.at[accr.at[sl]], rows.at[i & 1], gsem)

    gd = [fire(0), None]
    wd = [None, None]
    for i in range(len(items)):
        p = i & 1
        _, gh, sl = items[i]
        gd[p].wait()
        if i + 1 < len(items):
            q = (i + 1) & 1
            if wd[q] is not None:
                wd[q].wait()
                wd[q] = None
            gd[q] = fire(i + 1)
        wd[p] = pltpu.async_copy(
            rows.at[p], gh.at[sl, pl.ds(pbase, _CENT_PER_SUB)], wsem)
    for d in wd:
        if d is not None:
            d.wait()


_bq_call = functools.partial(
    pl.kernel, _bq_body,
    out_type=(jax.ShapeDtypeStruct((_NS0, _B * _NP, _CP), jnp.float32),
              jax.ShapeDtypeStruct((_NS1, _B * _NP, _CP), jnp.float32)),
    mesh=_mesh,
    scratch_types=[
        pltpu.VMEM((_N,), jnp.float32),
        pltpu.VMEM((_N,), jnp.float32),
        pltpu.VMEM((_N,), jnp.float32),
        pltpu.VMEM((_CENT_PER_SUB * _L,), jnp.float32),
        pltpu.VMEM((_NS0, _CENT_PER_SUB), jnp.int32),
        pltpu.VMEM((_NS1, _CENT_PER_SUB), jnp.int32),
        pltpu.VMEM((2, _CENT_PER_SUB, _CP), jnp.float32),
        pltpu.SemaphoreType.DMA,
        pltpu.SemaphoreType.DMA,
    ],
    compiler_params=_sc_params)()


_MGRID = 4
_S1B = _NS1 // _MGRID
_S0B = _NS0 // _MGRID


def _mlp_body(g0, g1, nx, w00, b00, w01, b01, w10, b10, w11, b11, o0, o1):
    s = pl.program_id(0)
    nxv = nx[...]

    def two_layer(g_ref, sl, w1, bb1, w2, bb2):
        h = g_ref[sl] - nxv
        a = jnp.dot(h, w1[...], preferred_element_type=jnp.float32) + bb1[...]
        a = jnp.maximum(a, 0.0)
        a = jnp.dot(a, w2[...], preferred_element_type=jnp.float32) + bb2[...]
        return jnp.maximum(a, 0.0)

    a1 = two_layer(g1, 0, w10, b10, w11, b11)
    for sl in range(1, _S1B):
        a1 = jnp.maximum(a1, two_layer(g1, sl, w10, b10, w11, b11))

    @pl.when(s == 0)
    def _():
        o1[...] = a1

    @pl.when(s > 0)
    def _():
        o1[...] = jnp.maximum(o1[...], a1)

    a0 = two_layer(g0, 0, w00, b00, w01, b01)
    for sl in range(1, _S0B):
        a0 = jnp.maximum(a0, two_layer(g0, sl, w00, b00, w01, b01))

    @pl.when(s == 0)
    def _():
        o0[...] = a0

    @pl.when(s > 0)
    def _():
        o0[...] = jnp.maximum(o0[...], a0)


def _mlp_call(g0, g1, nxyz24, w00, b00, w01, b01, w10, b10, w11, b11):
    r = _B * _NP
    full = lambda shape: pl.BlockSpec(shape, lambda s: (0,) * len(shape))
    return pl.pallas_call(
        _mlp_body,
        grid=(_MGRID,),
        in_specs=[
            pl.BlockSpec((_S0B, r, _CP), lambda s: (s, 0, 0)),
            pl.BlockSpec((_S1B, r, _CP), lambda s: (s, 0, 0)),
            full((r, _CP)),
            full((_CP, 32)), full((1, 32)),
            full((32, 32)), full((1, 32)),
            full((_CP, 32)), full((1, 32)),
            full((32, 64)), full((1, 64)),
        ],
        out_specs=[full((r, 32)), full((r, 64))],
        out_shape=[jax.ShapeDtypeStruct((r, 32), jnp.float32),
                   jax.ShapeDtypeStruct((r, 64), jnp.float32)],
    )(g0, g1, nxyz24, w00, b00, w01, b01, w10, b10, w11, b11)


def kernel(xyz, features, W0_0, b0_0, W0_1, b0_1, W1_0, b1_0, W1_1, b1_1):
    xs = xyz[:, :, 0].reshape(-1)
    ys = xyz[:, :, 1].reshape(-1)
    zs = xyz[:, :, 2].reshape(-1)
    cpad_flat = _fps_call(xs, ys, zs)
    cpad = cpad_flat.reshape(_B, _NP, _L)
    table = jnp.concatenate(
        [xyz, jnp.transpose(features, (0, 2, 1)),
         jnp.zeros((_B, _N, _CP - 3 - _CIN), jnp.float32)], axis=-1)
    table2d = table.reshape(_B * _N, _CP)
    g0, g1 = _bq_call(xs, ys, zs, cpad_flat, table2d)
    nxyz24 = jnp.pad(cpad.reshape(_B * _NP, _L), ((0, 0), (0, _CP - _L)))
    pad_w = lambda w: jnp.pad(w.T, ((0, _CP - w.shape[1]), (0, 0)))
    out0, out1 = _mlp_call(
        g0, g1, nxyz24,
        pad_w(W0_0), b0_0.reshape(1, -1), W0_1.T, b0_1.reshape(1, -1),
        pad_w(W1_0), b1_0.reshape(1, -1), W1_1.T, b1_1.reshape(1, -1))
    new_xyz = cpad[:, :, :3]
    nf0 = jnp.transpose(out0.reshape(_B, _NP, 32), (0, 2, 1))
    nf1 = jnp.transpose(out1.reshape(_B, _NP, 64), (0, 2, 1))
    new_features = jnp.concatenate([nf0, nf1], axis=1)
    return new_xyz, new_features


# speedup vs baseline: 1.2833x; 1.0048x over previous
import functools

import jax
import jax.numpy as jnp
import numpy as np
from jax import lax
from jax.experimental import pallas as pl
from jax.experimental.pallas import tpu as pltpu
from jax.experimental.pallas import tpu_sc as plsc

_B, _N, _CIN = 2, 16384, 16
_NP = 1024
_NS0, _NS1 = 16, 32
_R0SQ = float(np.float32(0.1) * np.float32(0.1))
_R1SQ = float(np.float32(0.2) * np.float32(0.2))
_CP = 24
_L = 16
_NSUB = 16
_NLOC = _N // _NSUB
_NCHUNK = _NLOC // _L
_BIG = 0x7FFFFFFF

_mesh = plsc.VectorSubcoreMesh(core_axis_name="c", subcore_axis_name="s",
                               num_cores=2, num_subcores=16)
_sc_params = pltpu.CompilerParams(needs_layout_passes=False,
                                  use_tc_tiling_on_sc=False)


def _fps_body(xs, ys, zs, cpad, xv, yv, zv, dv, rowv, bufv, accv, shared):
    c = lax.axis_index("c")
    s = lax.axis_index("s")
    base = s * _NLOC
    pltpu.sync_copy(xs.at[pl.ds(c * _N, _N)], xv)
    pltpu.sync_copy(ys.at[pl.ds(c * _N, _N)], yv)
    pltpu.sync_copy(zs.at[pl.ds(c * _N, _N)], zv)
    lanes = lax.iota(jnp.int32, _L)
    z16 = jnp.zeros((_L,), jnp.int32)
    ninf = jnp.float32(-1e30)

    def init_d(j, _):
        dv[pl.ds(j * _L, _L)] = jnp.full((_L,), 1e10, jnp.float32)
        return 0
    lax.fori_loop(0, _NCHUNK, init_d, 0)

    def publish_reduce(d_s, gi_s, par):
        row = jnp.where(lanes == 0, d_s,
              jnp.where(lanes == 1, gi_s.astype(jnp.float32), 0.0))
        rowv[...] = row
        pb = par * (_L * _L)
        pltpu.sync_copy(rowv, shared.at[pl.ds(pb + s * _L, _L)])
        plsc.subcore_barrier()
        pltpu.sync_copy(shared.at[pl.ds(pb, _L * _L)], bufv)
        tcol = lanes * _L
        dcol = plsc.load_gather(bufv, [tcol])
        gmax = jnp.max(dcol)
        wl = plsc.all_reduce_ffs(dcol == gmax)
        giv = plsc.load_gather(bufv, [wl * _L + 1]).astype(jnp.int32)
        cxw = plsc.load_gather(xv, [giv])
        cyw = plsc.load_gather(yv, [giv])
        czw = plsc.load_gather(zv, [giv])
        return cxw, cyw, czw

    d0 = jnp.where(s == 0, jnp.float32(1e30), ninf)
    carry0 = publish_reduce(d0, jnp.int32(0), jnp.int32(0))

    def body(i, carry):
        cx, cy, cz = carry

        @pl.when(s == 0)
        def _():
            rowc = jnp.where(lanes == 0, cx,
                   jnp.where(lanes == 1, cy,
                   jnp.where(lanes == 2, cz, 0.0)))
            plsc.store_scatter(accv, [i * _L + lanes], rowc)

        def chunk(j, ch):
            bd, bi = ch
            off = base + j * _L
            dx = xv[pl.ds(off, _L)] - cx
            dy = yv[pl.ds(off, _L)] - cy
            dz = zv[pl.ds(off, _L)] - cz
            d = (dx * dx + dy * dy) + dz * dz
            dmin = jnp.minimum(dv[pl.ds(j * _L, _L)], d)
            dv[pl.ds(j * _L, _L)] = dmin
            upd = dmin > bd
            bd = jnp.where(upd, dmin, bd)
            bi = jnp.where(upd, off + lanes, bi)
            return bd, bi

        bd, bi = plsc.parallel_loop(
            0, _NCHUNK, unroll=8,
            carry=(jnp.full((_L,), -1.0, jnp.float32),
                   jnp.zeros((_L,), jnp.int32)))(chunk)
        md = jnp.max(bd)
        gidx = jnp.min(jnp.where(bd == md, bi, _BIG))
        return publish_reduce(md, gidx, (i + 1) & 1)

    lax.fori_loop(0, _NP, body, carry0)

    @pl.when(s == 0)
    def _():
        pltpu.sync_copy(accv, cpad.at[pl.ds(c * _NP * _L, _NP * _L)])


_fps_call = functools.partial(
    pl.kernel, _fps_body,
    out_type=jax.ShapeDtypeStruct((_B * _NP * _L,), jnp.float32),
    mesh=_mesh,
    scratch_types=[
        pltpu.VMEM((_N,), jnp.float32),
        pltpu.VMEM((_N,), jnp.float32),
        pltpu.VMEM((_N,), jnp.float32),
        pltpu.VMEM((_NLOC,), jnp.float32),
        pltpu.VMEM((_L,), jnp.float32),
        pltpu.VMEM((_L * _L,), jnp.float32),
        pltpu.VMEM((_NP * _L,), jnp.float32),
        pltpu.VMEM_SHARED((2 * _L * _L,), jnp.float32),
    ],
    compiler_params=_sc_params)()


_GPTS = 128
_GCH = _GPTS // _L
_NGRP = _N // _GPTS
_CENT_PER_SUB = _NP // _NSUB


def _bq_body(xs, ys, zs, cpadf, table2d, g0, g1, xv, yv, zv, centv, acc0, acc1,
             rows, gsem, wsem):
    c = lax.axis_index("c")
    s = lax.axis_index("s")
    b = c
    pltpu.sync_copy(xs.at[pl.ds(b * _N, _N)], xv)
    pltpu.sync_copy(ys.at[pl.ds(b * _N, _N)], yv)
    pltpu.sync_copy(zs.at[pl.ds(b * _N, _N)], zv)
    cb = b * _NP * _L + s * _CENT_PER_SUB * _L
    pltpu.sync_copy(cpadf.at[pl.ds(cb, _CENT_PER_SUB * _L)], centv)
    lanes = lax.iota(jnp.int32, _L)
    z16 = jnp.zeros((_L,), jnp.int32)
    ninf = jnp.float32(-1e30)
    roff = b * _N

    def per_cent(ci, _):
        crow = centv[pl.ds(ci * _L, _L)]
        cx = jnp.max(jnp.where(lanes == 0, crow, ninf))
        cy = jnp.max(jnp.where(lanes == 1, crow, ninf))
        cz = jnp.max(jnp.where(lanes == 2, crow, ninf))

        def cond(st):
            g, c0, c1, notdone = st
            return (g < _NGRP) & notdone

        def wbody(st):
            g, c0, c1, notdone = st
            m0s, m1s = [], []
            for j in range(_GCH):
                off = g * _GPTS + j * _L
                dx = xv[pl.ds(off, _L)] - cx
                dy = yv[pl.ds(off, _L)] - cy
                dz = zv[pl.ds(off, _L)] - cz
                sq = (dx * dx + dy * dy) + dz * dz
                m0s.append(sq < _R0SQ)
                m1s.append(sq < _R1SQ)
            m1or = functools.reduce(jnp.bitwise_or, m1s)
            m0or = functools.reduce(jnp.bitwise_or, m0s)
            mm = (m1or & (c1 < _NS1)) | (m0or & (c0 < _NS0))

            def slow(_):
                cc0, cc1 = c0, c1
                for j in range(_GCH):
                    gv = (roff + g * _GPTS + j * _L) + lanes
                    p0 = cc0 + plsc.cumsum(m0s[j].astype(jnp.int32)) - 1
                    plsc.store_scatter(acc0, [p0, z16 + ci], gv,
                                       mask=m0s[j] & (p0 < _NS0))
                    cc0 = cc0 + plsc.all_reduce_population_count(m0s[j])
                    p1 = cc1 + plsc.cumsum(m1s[j].astype(jnp.int32)) - 1
                    plsc.store_scatter(acc1, [p1, z16 + ci], gv,
                                       mask=m1s[j] & (p1 < _NS1))
                    cc1 = cc1 + plsc.all_reduce_population_count(m1s[j])
                nd = jnp.any((cc0 < _NS0) | (cc1 < _NS1))
                return cc0, cc1, nd

            c0n, c1n, ndn = lax.cond(jnp.any(mm), slow,
                                     lambda _: (c0, c1, notdone), 0)
            return g + jnp.int32(1), c0n, c1n, ndn

        zc = jnp.zeros((_L,), jnp.int32)
        g, c0, c1, _nd = lax.while_loop(
            cond, wbody, (jnp.int32(0), zc, zc, jnp.bool_(True)))
        civ = z16 + ci
        f0 = plsc.load_gather(acc0, [z16, civ])
        f0 = jnp.where(c0 > 0, f0, roff)
        plsc.store_scatter(acc0, [lanes, civ], f0, mask=lanes >= c0)
        f1 = plsc.load_gather(acc1, [z16, civ])
        f1 = jnp.where(c1 > 0, f1, roff)
        plsc.store_scatter(acc1, [lanes, civ], f1, mask=lanes >= c1)
        plsc.store_scatter(acc1, [lanes + _L, civ], f1, mask=(lanes + _L) >= c1)
        return 0

    lax.fori_loop(0, _CENT_PER_SUB, per_cent, 0)

    pbase = b * _NP + s * _CENT_PER_SUB

    items = [(acc0, g0, sl) for sl in range(_NS0)] \
          + [(acc1, g1, sl) for sl in range(_NS1)]

    def fire(i):
        accr, gh, sl = items[i]
        return pltpu.async_copy(table2d.at[accr.at[sl]], rows.at[i & 1], gsem)

    gd = [fire(0), None]
    wd = [None, None]
    for i in range(len(items)):
        p = i & 1
        _, gh, sl = items[i]
        gd[p].wait()
        if i + 1 < len(items):
            q = (i + 1) & 1
            if wd[q] is not None:
                wd[q].wait()
                wd[q] = None
            gd[q] = fire(i + 1)
        wd[p] = pltpu.async_copy(
            rows.at[p], gh.at[sl, pl.ds(pbase, _CENT_PER_SUB)], wsem)
    for d in wd:
        if d is not None:
            d.wait()


_bq_call = functools.partial(
    pl.kernel, _bq_body,
    out_type=(jax.ShapeDtypeStruct((_NS0, _B * _NP, _CP), jnp.float32),
              jax.ShapeDtypeStruct((_NS1, _B * _NP, _CP), jnp.float32)),
    mesh=_mesh,
    scratch_types=[
        pltpu.VMEM((_N,), jnp.float32),
        pltpu.VMEM((_N,), jnp.float32),
        pltpu.VMEM((_N,), jnp.float32),
        pltpu.VMEM((_CENT_PER_SUB * _L,), jnp.float32),
        pltpu.VMEM((_NS0, _CENT_PER_SUB), jnp.int32),
        pltpu.VMEM((_NS1, _CENT_PER_SUB), jnp.int32),
        pltpu.VMEM((2, _CENT_PER_SUB, _CP), jnp.float32),
        pltpu.SemaphoreType.DMA,
        pltpu.SemaphoreType.DMA,
    ],
    compiler_params=_sc_params)()


_MGRID = 4
_S1B = _NS1 // _MGRID
_S0B = _NS0 // _MGRID


def _mlp_body(g0, g1, nx, w00, b00, w01, b01, w10, b10, w11, b11, o0, o1):
    s = pl.program_id(0)
    nxv = nx[...]

    def two_layer(g_ref, sl, w1, bb1, w2, bb2):
        h = g_ref[sl] - nxv
        a = jnp.dot(h, w1[...], preferred_element_type=jnp.float32) + bb1[...]
        a = jnp.maximum(a, 0.0)
        a = jnp.dot(a, w2[...], preferred_element_type=jnp.float32) + bb2[...]
        return jnp.maximum(a, 0.0)

    a1 = two_layer(g1, 0, w10, b10, w11, b11)
    for sl in range(1, _S1B):
        a1 = jnp.maximum(a1, two_layer(g1, sl, w10, b10, w11, b11))

    @pl.when(s == 0)
    def _():
        o1[...] = a1

    @pl.when(s > 0)
    def _():
        o1[...] = jnp.maximum(o1[...], a1)

    a0 = two_layer(g0, 0, w00, b00, w01, b01)
    for sl in range(1, _S0B):
        a0 = jnp.maximum(a0, two_layer(g0, sl, w00, b00, w01, b01))

    @pl.when(s == 0)
    def _():
        o0[...] = a0

    @pl.when(s > 0)
    def _():
        o0[...] = jnp.maximum(o0[...], a0)


def _mlp_call(g0, g1, nxyz24, w00, b00, w01, b01, w10, b10, w11, b11):
    r = _B * _NP
    full = lambda shape: pl.BlockSpec(shape, lambda s: (0,) * len(shape))
    return pl.pallas_call(
        _mlp_body,
        grid=(_MGRID,),
        in_specs=[
            pl.BlockSpec((_S0B, r, _CP), lambda s: (s, 0, 0)),
            pl.BlockSpec((_S1B, r, _CP), lambda s: (s, 0, 0)),
            full((r, _CP)),
            full((_CP, 32)), full((1, 32)),
            full((32, 32)), full((1, 32)),
            full((_CP, 32)), full((1, 32)),
            full((32, 64)), full((1, 64)),
        ],
        out_specs=[full((r, 32)), full((r, 64))],
        out_shape=[jax.ShapeDtypeStruct((r, 32), jnp.float32),
                   jax.ShapeDtypeStruct((r, 64), jnp.float32)],
    )(g0, g1, nxyz24, w00, b00, w01, b01, w10, b10, w11, b11)


def kernel(xyz, features, W0_0, b0_0, W0_1, b0_1, W1_0, b1_0, W1_1, b1_1):
    xs = xyz[:, :, 0].reshape(-1)
    ys = xyz[:, :, 1].reshape(-1)
    zs = xyz[:, :, 2].reshape(-1)
    cpad_flat = _fps_call(xs, ys, zs)
    cpad = cpad_flat.reshape(_B, _NP, _L)
    table = jnp.concatenate(
        [xyz, jnp.transpose(features, (0, 2, 1)),
         jnp.zeros((_B, _N, _CP - 3 - _CIN), jnp.float32)], axis=-1)
    table2d = table.reshape(_B * _N, _CP)
    g0, g1 = _bq_call(xs, ys, zs, cpad_flat, table2d)
    nxyz24 = jnp.pad(cpad.reshape(_B * _NP, _L), ((0, 0), (0, _CP - _L)))
    pad_w = lambda w: jnp.pad(w.T, ((0, _CP - w.shape[1]), (0, 0)))
    out0, out1 = _mlp_call(
        g0, g1, nxyz24,
        pad_w(W0_0), b0_0.reshape(1, -1), W0_1.T, b0_1.reshape(1, -1),
        pad_w(W1_0), b1_0.reshape(1, -1), W1_1.T, b1_1.reshape(1, -1))
    new_xyz = cpad[:, :, :3]
    nf0 = jnp.transpose(out0.reshape(_B, _NP, 32), (0, 2, 1))
    nf1 = jnp.transpose(out1.reshape(_B, _NP, 64), (0, 2, 1))
    new_features = jnp.concatenate([nf0, nf1], axis=1)
    return new_xyz, new_features
